# baseline (speedup 1.0000x reference)
import functools

import jax
import jax.numpy as jnp
from jax import lax
from jax.experimental import pallas as pl
from jax.experimental.pallas import tpu as pltpu
from jax.experimental.pallas import tpu_sc as plsc

_OUT_DIM = 64
_PAD_DIM = 128
_TAB_ROWS = 5128

_NC = 2
_NS = 16
_NW = _NC * _NS

_CHUNK = 400


def _table_body(pos_emb_ref, w_ref, b_ref, t_ref):
    t = lax.dot_general(
        pos_emb_ref[...], w_ref[...],
        dimension_numbers=(((1,), (1,)), ((), ())),
        preferred_element_type=jnp.float32,
    ) + b_ref[...]
    t_ref[...] = jnp.pad(t, ((0, _TAB_ROWS - t.shape[0]), (0, _PAD_DIM - t.shape[1])))


def _make_table(pos_emb, W, b):
    return pl.pallas_call(
        _table_body,
        out_shape=jax.ShapeDtypeStruct((_TAB_ROWS, _PAD_DIM), jnp.float32),
    )(pos_emb, W, b.reshape(1, _OUT_DIM))


def _gather_body(n_per_w, n_chunks, hist, table_hbm, idx_hbm, out_hbm,
                 idx_all, g, c0, c1, sg, sw0, sw1):
    cid = lax.axis_index("c")
    sid = lax.axis_index("s")
    wid = sid * _NC + cid
    base = wid * n_per_w
    bat = _CHUNK // hist
    hh = hist // 2

    pltpu.sync_copy(idx_hbm.at[pl.ds(base, n_per_w)], idx_all)

    def gather_pair(i):
        idx_s = idx_all.at[pl.ds(i * _CHUNK, _CHUNK)]
        return table_hbm.at[idx_s], g

    def start_gather(i):
        src, dst = gather_pair(i)
        pltpu.async_copy(src, dst, sg)

    def wait_gather(i):
        src, dst = gather_pair(i)
        pltpu.make_async_copy(src, dst, sg).wait()

    def bridge(c):
        def row(r, carry):
            for half in range(2):
                for cc in range(_OUT_DIM // 16):
                    c[r // hh, r % hh, pl.ds(half * _OUT_DIM + cc * 16, 16)] = (
                        g[2 * r + half, pl.ds(cc * 16, 16)])
            return carry
        lax.fori_loop(0, _CHUNK // 2, row, 0, unroll=4)

    def out_slice(i):
        return out_hbm.at[pl.ds((base + i * _CHUNK) // hist, bat)]

    start_gather(0)

    def pair(j, carry):
        i0 = j * 2

        wait_gather(i0)

        @pl.when(j > 0)
        def _():
            pltpu.make_async_copy(c0, out_slice(0), sw0).wait()

        bridge(c0)
        start_gather(i0 + 1)
        pltpu.async_copy(c0, out_slice(i0), sw0)

        wait_gather(i0 + 1)

        @pl.when(j > 0)
        def _():
            pltpu.make_async_copy(c1, out_slice(1), sw1).wait()

        bridge(c1)

        @pl.when(j < n_chunks // 2 - 1)
        def _():
            start_gather(i0 + 2)

        pltpu.async_copy(c1, out_slice(i0 + 1), sw1)
        return carry

    lax.fori_loop(0, n_chunks // 2, pair, 0)

    pltpu.make_async_copy(c0, out_slice(0), sw0).wait()
    pltpu.make_async_copy(c1, out_slice(1), sw1).wait()


def _make_gather(batch, hist):
    n_total = batch * hist
    n_per_w = n_total // _NW
    n_chunks = n_per_w // _CHUNK
    bat = _CHUNK // hist
    assert n_chunks % 2 == 0 and _CHUNK % hist == 0
    mesh = plsc.VectorSubcoreMesh(core_axis_name="c", subcore_axis_name="s")
    return functools.partial(
        pl.kernel,
        mesh=mesh,
        out_type=jax.ShapeDtypeStruct((batch, hist // 2, _PAD_DIM), jnp.float32),
        scratch_types=[
            pltpu.VMEM((n_per_w,), jnp.int32),
            pltpu.VMEM((_CHUNK, _PAD_DIM), jnp.float32),
            pltpu.VMEM((bat, hist // 2, _PAD_DIM), jnp.float32),
            pltpu.VMEM((bat, hist // 2, _PAD_DIM), jnp.float32),
            pltpu.SemaphoreType.DMA,
            pltpu.SemaphoreType.DMA,
            pltpu.SemaphoreType.DMA,
        ],
    )(functools.partial(_gather_body, n_per_w, n_chunks, hist))


def kernel(positions, pos_emb, W, b):
    batch, hist = positions.shape
    n_total = batch * hist
    table = _make_table(pos_emb, W, b)
    idx = positions.reshape(n_total).astype(jnp.int32)
    out2 = _make_gather(batch, hist)(table, idx)
    return out2.reshape(batch, hist, _OUT_DIM)


# speedup vs baseline: 1.3758x; 1.3758x over previous
import functools

import jax
import jax.numpy as jnp
from jax import lax
from jax.experimental import pallas as pl
from jax.experimental.pallas import tpu as pltpu
from jax.experimental.pallas import tpu_sc as plsc

_OUT_DIM = 64
_PAD_DIM = 128
_TAB_ROWS = 5128

_NC = 2
_NS = 16
_NW = _NC * _NS

_CHUNK = 200


def _table_body(pos_emb_ref, w_ref, b_ref, t_ref):
    t = lax.dot_general(
        pos_emb_ref[...], w_ref[...],
        dimension_numbers=(((1,), (1,)), ((), ())),
        preferred_element_type=jnp.float32,
    ) + b_ref[...]
    t_ref[...] = jnp.pad(t, ((0, _TAB_ROWS - t.shape[0]), (0, _PAD_DIM - t.shape[1])))


def _make_table(pos_emb, W, b):
    return pl.pallas_call(
        _table_body,
        out_shape=jax.ShapeDtypeStruct((_TAB_ROWS, _PAD_DIM), jnp.float32),
    )(pos_emb, W, b.reshape(1, _OUT_DIM))


def _gather_body(n_per_w, n_chunks, hist, table_hbm, idx_hbm, out_hbm,
                 idx_all, g0, g1, c0, c1, sg0, sg1, sw0, sw1):
    cid = lax.axis_index("c")
    sid = lax.axis_index("s")
    wid = sid * _NC + cid
    base = wid * n_per_w

    pltpu.sync_copy(idx_hbm.at[pl.ds(base, n_per_w)], idx_all)

    def gather_pair(i, g):
        idx_s = idx_all.at[pl.ds(i * _CHUNK, _CHUNK)]
        return table_hbm.at[idx_s], g

    def start_gather(i, g, sem):
        src, dst = gather_pair(i, g)
        pltpu.async_copy(src, dst, sem)

    def wait_gather(i, g, sem):
        src, dst = gather_pair(i, g)
        pltpu.make_async_copy(src, dst, sem).wait()

    def bridge(g, c):
        def row(r, carry):
            for half in range(2):
                for cc in range(_OUT_DIM // 16):
                    c[r, pl.ds(half * _OUT_DIM + cc * 16, 16)] = (
                        g[2 * r + half, pl.ds(cc * 16, 16)])
            return carry
        lax.fori_loop(0, _CHUNK // 2, row, 0, unroll=4)

    def out_slice(i):
        return out_hbm.at[(base + i * _CHUNK) // hist]

    start_gather(0, g0, sg0)
    start_gather(1, g1, sg1)

    def pair(j, carry):
        i0 = j * 2

        @pl.when(j > 0)
        def _():
            pltpu.make_async_copy(c0, out_slice(0), sw0).wait()

        wait_gather(i0, g0, sg0)
        bridge(g0, c0)
        pltpu.async_copy(c0, out_slice(i0), sw0)

        @pl.when(j < n_chunks // 2 - 1)
        def _():
            start_gather(i0 + 2, g0, sg0)

        @pl.when(j > 0)
        def _():
            pltpu.make_async_copy(c1, out_slice(1), sw1).wait()

        wait_gather(i0 + 1, g1, sg1)
        bridge(g1, c1)
        pltpu.async_copy(c1, out_slice(i0 + 1), sw1)

        @pl.when(j < n_chunks // 2 - 1)
        def _():
            start_gather(i0 + 3, g1, sg1)

        return carry

    lax.fori_loop(0, n_chunks // 2, pair, 0)

    pltpu.make_async_copy(c0, out_slice(0), sw0).wait()
    pltpu.make_async_copy(c1, out_slice(1), sw1).wait()


def _make_gather(batch, hist):
    n_total = batch * hist
    n_per_w = n_total // _NW
    n_chunks = n_per_w // _CHUNK
    assert n_chunks % 2 == 0 and _CHUNK == hist
    mesh = plsc.VectorSubcoreMesh(core_axis_name="c", subcore_axis_name="s")
    return functools.partial(
        pl.kernel,
        mesh=mesh,
        out_type=jax.ShapeDtypeStruct((batch, hist // 2, _PAD_DIM), jnp.float32),
        scratch_types=[
            pltpu.VMEM((n_per_w,), jnp.int32),
            pltpu.VMEM((_CHUNK, _PAD_DIM), jnp.float32),
            pltpu.VMEM((_CHUNK, _PAD_DIM), jnp.float32),
            pltpu.VMEM((_CHUNK // 2, _PAD_DIM), jnp.float32),
            pltpu.VMEM((_CHUNK // 2, _PAD_DIM), jnp.float32),
            pltpu.SemaphoreType.DMA,
            pltpu.SemaphoreType.DMA,
            pltpu.SemaphoreType.DMA,
            pltpu.SemaphoreType.DMA,
        ],
    )(functools.partial(_gather_body, n_per_w, n_chunks, hist))


def kernel(positions, pos_emb, W, b):
    batch, hist = positions.shape
    n_total = batch * hist
    table = _make_table(pos_emb, W, b)
    idx = positions.reshape(n_total).astype(jnp.int32)
    out2 = _make_gather(batch, hist)(table, idx)
    return out2.reshape(batch, hist, _OUT_DIM)


# speedup vs baseline: 1.3784x; 1.0019x over previous
import functools

import jax
import jax.numpy as jnp
from jax import lax
from jax.experimental import pallas as pl
from jax.experimental.pallas import tpu as pltpu
from jax.experimental.pallas import tpu_sc as plsc

_OUT_DIM = 64
_PAD_DIM = 128
_TAB_ROWS = 5128

_NC = 2
_NS = 16
_NW = _NC * _NS

_CHUNK = 200


def _table_body(pos_emb_ref, w_ref, b_ref, t_ref):
    t = lax.dot_general(
        pos_emb_ref[...], w_ref[...],
        dimension_numbers=(((1,), (1,)), ((), ())),
        preferred_element_type=jnp.float32,
    ) + b_ref[...]
    t_ref[...] = jnp.pad(t, ((0, _TAB_ROWS - t.shape[0]), (0, _PAD_DIM - t.shape[1])))


def _make_table(pos_emb, W, b):
    return pl.pallas_call(
        _table_body,
        out_shape=jax.ShapeDtypeStruct((_TAB_ROWS, _PAD_DIM), jnp.float32),
    )(pos_emb, W, b.reshape(1, _OUT_DIM))


def _gather_body(n_per_w, n_chunks, hist, table_hbm, idx_hbm, out_hbm,
                 idx_all, g0, g1, c0, c1, sg0, sg1, sw0, sw1):
    cid = lax.axis_index("c")
    sid = lax.axis_index("s")
    wid = sid * _NC + cid
    base = wid * n_per_w

    pltpu.sync_copy(idx_hbm.at[pl.ds(base, n_per_w)], idx_all)

    def gather_pair(i, g):
        idx_s = idx_all.at[pl.ds(i * _CHUNK, _CHUNK)]
        return table_hbm.at[idx_s], g

    def start_gather(i, g, sem):
        src, dst = gather_pair(i, g)
        pltpu.async_copy(src, dst, sem)

    def wait_gather(i, g, sem):
        src, dst = gather_pair(i, g)
        pltpu.make_async_copy(src, dst, sem).wait()

    def bridge(g, c):
        def row(r, carry):
            for half in range(2):
                for cc in range(_OUT_DIM // 16):
                    c[r, pl.ds(half * _OUT_DIM + cc * 16, 16)] = (
                        g[2 * r + half, pl.ds(cc * 16, 16)])
            return carry
        lax.fori_loop(0, _CHUNK // 2, row, 0, unroll=10)

    def out_slice(i):
        return out_hbm.at[(base + i * _CHUNK) // hist]

    start_gather(0, g0, sg0)
    start_gather(1, g1, sg1)

    def pair(j, carry):
        i0 = j * 2

        @pl.when(j > 0)
        def _():
            pltpu.make_async_copy(c0, out_slice(0), sw0).wait()

        wait_gather(i0, g0, sg0)
        bridge(g0, c0)
        pltpu.async_copy(c0, out_slice(i0), sw0)

        @pl.when(j < n_chunks // 2 - 1)
        def _():
            start_gather(i0 + 2, g0, sg0)

        @pl.when(j > 0)
        def _():
            pltpu.make_async_copy(c1, out_slice(1), sw1).wait()

        wait_gather(i0 + 1, g1, sg1)
        bridge(g1, c1)
        pltpu.async_copy(c1, out_slice(i0 + 1), sw1)

        @pl.when(j < n_chunks // 2 - 1)
        def _():
            start_gather(i0 + 3, g1, sg1)

        return carry

    lax.fori_loop(0, n_chunks // 2, pair, 0)

    pltpu.make_async_copy(c0, out_slice(0), sw0).wait()
    pltpu.make_async_copy(c1, out_slice(1), sw1).wait()


def _make_gather(batch, hist):
    n_total = batch * hist
    n_per_w = n_total // _NW
    n_chunks = n_per_w // _CHUNK
    assert n_chunks % 2 == 0 and _CHUNK == hist
    mesh = plsc.VectorSubcoreMesh(core_axis_name="c", subcore_axis_name="s")
    return functools.partial(
        pl.kernel,
        mesh=mesh,
        out_type=jax.ShapeDtypeStruct((batch, hist // 2, _PAD_DIM), jnp.float32),
        scratch_types=[
            pltpu.VMEM((n_per_w,), jnp.int32),
            pltpu.VMEM((_CHUNK, _PAD_DIM), jnp.float32),
            pltpu.VMEM((_CHUNK, _PAD_DIM), jnp.float32),
            pltpu.VMEM((_CHUNK // 2, _PAD_DIM), jnp.float32),
            pltpu.VMEM((_CHUNK // 2, _PAD_DIM), jnp.float32),
            pltpu.SemaphoreType.DMA,
            pltpu.SemaphoreType.DMA,
            pltpu.SemaphoreType.DMA,
            pltpu.SemaphoreType.DMA,
        ],
    )(functools.partial(_gather_body, n_per_w, n_chunks, hist))


def kernel(positions, pos_emb, W, b):
    batch, hist = positions.shape
    n_total = batch * hist
    table = _make_table(pos_emb, W, b)
    idx = positions.reshape(n_total).astype(jnp.int32)
    out2 = _make_gather(batch, hist)(table, idx)
    return out2.reshape(batch, hist, _OUT_DIM)
